# baseline (speedup 1.0000x reference)
import jax
import jax.numpy as jnp
import numpy as np
from jax.experimental import pallas as pl
from jax.experimental.pallas import tpu as pltpu

_E = 64
_MIN_K, _MAX_K = 1, 4
_TOK_BLOCK = 4096
_SUB = 1024


def _router_kernel(xa_ref, xb_ref, cw_ref, rb_ref, b1_ref, w2_ref, b2_ref,
                   wts_ref, idx_ref, k_ref):
    half = _TOK_BLOCK // 2
    for s in range(half // _SUB):
        _router_slab(xa_ref, cw_ref, rb_ref, b1_ref, w2_ref, b2_ref,
                     wts_ref, idx_ref, k_ref, s * _SUB, 0)
    for s in range(half // _SUB):
        _router_slab(xb_ref, cw_ref, rb_ref, b1_ref, w2_ref, b2_ref,
                     wts_ref, idx_ref, k_ref, s * _SUB, half)


def _router_slab(x_ref, cw_ref, rb_ref, b1_ref, w2_ref, b2_ref,
                 wts_ref, idx_ref, k_ref, base, out_off):
    x = x_ref[pl.ds(base, _SUB), :]
    h4 = b1_ref.shape[1]

    comb = jax.lax.dot_general(
        x, cw_ref[...], (((1,), (1,)), ((), ())),
        preferred_element_type=jnp.float32)

    u_hid = comb[:, :h4] + b1_ref[...]
    u_hid = 0.5 * u_hid * (1.0 + jax.lax.erf(u_hid * np.float32(0.7071067811865476)))
    u = jnp.dot(u_hid, w2_ref[...], preferred_element_type=jnp.float32)
    u = u + b2_ref[...]
    u = jax.nn.sigmoid(u)
    k_float = _MIN_K + (_MAX_K - _MIN_K) * u
    k = jnp.clip(jnp.round(k_float).astype(jnp.int32), _MIN_K, _MAX_K)

    logits = comb[:, h4:] + rb_ref[...]

    t = logits.shape[0]
    lt = logits.T
    erow = jax.lax.broadcasted_iota(jnp.int32, (_E, t), 0)
    vals = []
    args = []
    for _ in range(_MAX_K):
        m = jnp.max(lt, axis=0, keepdims=True)
        a = jnp.min(jnp.where(lt == m, erow, _E), axis=0, keepdims=True)
        vals.append(m)
        args.append(a)
        lt = jnp.where(erow == a, -jnp.inf, lt)
    top_v = jnp.concatenate(vals, axis=0)
    top_i = jnp.concatenate(args, axis=0)

    kt = k.T
    pos = jax.lax.broadcasted_iota(jnp.int32, (_MAX_K, t), 0)
    mask = pos < kt
    w = jnp.where(mask, top_v, 0.0)
    w_max = jnp.max(w, axis=0, keepdims=True)
    e = jnp.exp(w - w_max)
    o = out_off + base
    wts_ref[:, pl.ds(o, _SUB)] = e / jnp.sum(e, axis=0, keepdims=True)
    idx_ref[:, pl.ds(o, _SUB)] = jnp.where(mask, top_i, -1)
    k_ref[pl.ds(o, _SUB)] = kt.reshape(kt.shape[1])


def kernel(hidden_states, router_W, router_b, u_W1, u_b1, u_W2, u_b2):
    B, S, D = hidden_states.shape
    N = B * S
    H4 = u_W1.shape[0]
    x = hidden_states.reshape(N, D)
    grid = (N // _TOK_BLOCK,)

    comb_W = jnp.concatenate([u_W1, router_W], axis=0).astype(jnp.bfloat16)

    full = lambda shape: pl.BlockSpec(shape, lambda i: (0, 0))

    wts, idx, k = pl.pallas_call(
        _router_kernel,
        grid=grid,
        in_specs=[
            pl.BlockSpec((_TOK_BLOCK // 2, D), lambda i: (2 * i, 0)),
            pl.BlockSpec((_TOK_BLOCK // 2, D), lambda i: (2 * i + 1, 0)),
            full((H4 + _E, D)),
            full((1, _E)),
            full((1, H4)),
            full((H4, 1)),
            full((1, 1)),
        ],
        out_specs=[
            pl.BlockSpec((_MAX_K, _TOK_BLOCK), lambda i: (0, i)),
            pl.BlockSpec((_MAX_K, _TOK_BLOCK), lambda i: (0, i)),
            pl.BlockSpec((_TOK_BLOCK,), lambda i: (i,)),
        ],
        out_shape=[
            jax.ShapeDtypeStruct((_MAX_K, N), jnp.float32),
            jax.ShapeDtypeStruct((_MAX_K, N), jnp.int32),
            jax.ShapeDtypeStruct((N,), jnp.int32),
        ],
        compiler_params=pltpu.CompilerParams(
            dimension_semantics=("parallel",),
        ),
    )(
        x,
        x,
        comb_W,
        router_b.reshape(1, _E),
        u_b1.reshape(1, H4),
        u_W2.reshape(H4, 1).astype(jnp.bfloat16),
        u_b2.reshape(1, 1),
    )
    return (wts.reshape(_MAX_K, B, S).transpose(1, 2, 0),
            idx.reshape(_MAX_K, B, S).transpose(1, 2, 0),
            k.reshape(B, S))


# speedup vs baseline: 1.0092x; 1.0092x over previous
import jax
import jax.numpy as jnp
import numpy as np
from jax.experimental import pallas as pl
from jax.experimental.pallas import tpu as pltpu

_E = 64
_MIN_K, _MAX_K = 1, 4
_TOK_BLOCK = 4096
_SUB = 1024


def _router_kernel(x_ref, cw_ref, rb_ref, b1_ref, w2_ref, b2_ref,
                   wts_ref, idx_ref, k_ref):
    for s in range(_TOK_BLOCK // _SUB):
        _router_slab(x_ref, cw_ref, rb_ref, b1_ref, w2_ref, b2_ref,
                     wts_ref, idx_ref, k_ref, s * _SUB)


def _router_slab(x_ref, cw_ref, rb_ref, b1_ref, w2_ref, b2_ref,
                 wts_ref, idx_ref, k_ref, base):
    x = x_ref[pl.ds(base, _SUB), :]
    h4 = b1_ref.shape[1]

    comb = jax.lax.dot_general(
        x, cw_ref[...], (((1,), (1,)), ((), ())),
        preferred_element_type=jnp.float32)

    u_hid = comb[:, :h4] + b1_ref[...]
    u_hid = 0.5 * u_hid * (1.0 + jax.lax.erf(u_hid * np.float32(0.7071067811865476)))
    u = jnp.dot(u_hid, w2_ref[...], preferred_element_type=jnp.float32)
    u = u + b2_ref[...]
    u = jax.nn.sigmoid(u)
    k_float = _MIN_K + (_MAX_K - _MIN_K) * u
    k = jnp.clip(jnp.round(k_float).astype(jnp.int32), _MIN_K, _MAX_K)

    logits = comb[:, h4:] + rb_ref[...]

    t = logits.shape[0]
    lt = logits.T
    erow = jax.lax.broadcasted_iota(jnp.int32, (_E, t), 0)
    vals = []
    args = []
    for _ in range(_MAX_K):
        m = jnp.max(lt, axis=0, keepdims=True)
        a = jnp.min(jnp.where(lt == m, erow, _E), axis=0, keepdims=True)
        vals.append(m)
        args.append(a)
        lt = jnp.where(erow == a, -jnp.inf, lt)
    top_v = jnp.concatenate(vals, axis=0)
    top_i = jnp.concatenate(args, axis=0)

    kt = k.T
    pos = jax.lax.broadcasted_iota(jnp.int32, (_MAX_K, t), 0)
    mask = pos < kt
    w = jnp.where(mask, top_v, 0.0)
    w_max = jnp.max(w, axis=0, keepdims=True)
    e = jnp.exp(w - w_max)
    wts_ref[:, pl.ds(base, _SUB)] = e / jnp.sum(e, axis=0, keepdims=True)
    idx_ref[:, pl.ds(base, _SUB)] = jnp.where(mask, top_i, -1)
    k_ref[pl.ds(base, _SUB)] = kt.reshape(kt.shape[1])


def kernel(hidden_states, router_W, router_b, u_W1, u_b1, u_W2, u_b2):
    B, S, D = hidden_states.shape
    N = B * S
    H4 = u_W1.shape[0]
    x = hidden_states.reshape(N, D)
    grid = (N // _TOK_BLOCK,)

    comb_W = jnp.concatenate([u_W1, router_W], axis=0).astype(jnp.bfloat16)

    full = lambda shape: pl.BlockSpec(shape, lambda i: (0, 0))

    wts, idx, k = pl.pallas_call(
        _router_kernel,
        grid=grid,
        in_specs=[
            pl.BlockSpec((_TOK_BLOCK, D), lambda i: (i, 0)),
            full((H4 + _E, D)),
            full((1, _E)),
            full((1, H4)),
            full((H4, 1)),
            full((1, 1)),
        ],
        out_specs=[
            pl.BlockSpec((_MAX_K, _TOK_BLOCK), lambda i: (0, i)),
            pl.BlockSpec((_MAX_K, _TOK_BLOCK), lambda i: (0, i)),
            pl.BlockSpec((_TOK_BLOCK,), lambda i: (i,)),
        ],
        out_shape=[
            jax.ShapeDtypeStruct((_MAX_K, N), jnp.float32),
            jax.ShapeDtypeStruct((_MAX_K, N), jnp.int32),
            jax.ShapeDtypeStruct((N,), jnp.int32),
        ],
        compiler_params=pltpu.CompilerParams(
            dimension_semantics=("arbitrary",),
        ),
    )(
        x,
        comb_W,
        router_b.reshape(1, _E),
        u_b1.reshape(1, H4),
        u_W2.reshape(H4, 1).astype(jnp.bfloat16),
        u_b2.reshape(1, 1),
    )
    return (wts.reshape(_MAX_K, B, S).transpose(1, 2, 0),
            idx.reshape(_MAX_K, B, S).transpose(1, 2, 0),
            k.reshape(B, S))
